# baseline (device time: 183602 ns/iter reference)
import jax
import jax.numpy as jnp
from jax import lax
from jax.experimental import pallas as pl
from jax.experimental.pallas import tpu as pltpu

N_DEV = 8
TOKENS = 1024
D_MODEL = 256
D_HIDDEN = 512
E_GLOBAL = 32
E_LOCAL = E_GLOBAL // N_DEV
CAP = 25


def kernel(x, router_W, route_idx, expert_W):
    del router_W

    def body(x_ref, ridx_ref, ew_ref, out_ref, comm_ref, send_sems, recv_sems):
        my = lax.axis_index("i")
        left = (my - 1) % N_DEV
        right = (my + 1) % N_DEV

        barrier_sem = pltpu.get_barrier_semaphore()
        for nbr in (left, right):
            pl.semaphore_signal(
                barrier_sem, inc=1,
                device_id=(nbr,), device_id_type=pl.DeviceIdType.MESH,
            )
        pl.semaphore_wait(barrier_sem, 2)

        ridx = ridx_ref[:, :]
        eids = lax.broadcasted_iota(jnp.int32, (TOKENS, E_GLOBAL), 1)
        onehot = (ridx == eids).astype(jnp.float32)
        row = lax.broadcasted_iota(jnp.int32, (TOKENS, TOKENS), 0)
        col = lax.broadcasted_iota(jnp.int32, (TOKENS, TOKENS), 1)
        tril = (col < row).astype(jnp.float32)
        prior = jnp.dot(tril, onehot, preferred_element_type=jnp.float32)
        rank = jnp.sum(prior * onehot, axis=1, keepdims=True)
        keep = (rank < CAP).astype(jnp.float32)

        acc = jnp.zeros((TOKENS, D_HIDDEN), jnp.float32)
        for k in range(E_LOCAL):
            e = my * E_LOCAL + k
            mask = jnp.where(ridx == e, keep, 0.0)
            acc = acc + jnp.dot(
                x_ref[:, :] * mask, ew_ref[k],
                preferred_element_type=jnp.float32,
            )
        out_ref[:, :] = acc
        comm_ref[0, :, :] = acc

        for h in range(N_DEV - 1):
            rdma = pltpu.make_async_remote_copy(
                src_ref=comm_ref.at[h],
                dst_ref=comm_ref.at[h + 1],
                send_sem=send_sems.at[h],
                recv_sem=recv_sems.at[h],
                device_id=(right,),
                device_id_type=pl.DeviceIdType.MESH,
            )
            rdma.start()
            rdma.wait()
            out_ref[:, :] += comm_ref[h + 1, :, :]

    return pl.pallas_call(
        body,
        out_shape=jax.ShapeDtypeStruct((TOKENS, D_HIDDEN), jnp.float32),
        in_specs=[
            pl.BlockSpec(memory_space=pltpu.VMEM),
            pl.BlockSpec(memory_space=pltpu.VMEM),
            pl.BlockSpec(memory_space=pltpu.VMEM),
        ],
        out_specs=pl.BlockSpec(memory_space=pltpu.VMEM),
        scratch_shapes=[
            pltpu.VMEM((N_DEV, TOKENS, D_HIDDEN), jnp.float32),
            pltpu.SemaphoreType.DMA((N_DEV - 1,)),
            pltpu.SemaphoreType.DMA((N_DEV - 1,)),
        ],
        compiler_params=pltpu.CompilerParams(collective_id=0),
    )(x, route_idx, expert_W)


# device time: 47764 ns/iter; 3.8439x vs baseline; 3.8439x over previous
import jax
import jax.numpy as jnp
from jax import lax
from jax.experimental import pallas as pl
from jax.experimental.pallas import tpu as pltpu

N_DEV = 8
TOKENS = 1024
D_MODEL = 256
D_HIDDEN = 512
E_GLOBAL = 32
E_LOCAL = E_GLOBAL // N_DEV
CAP = 25
SLOT = 32
CROWS = E_LOCAL * SLOT


def kernel(x, router_W, route_idx, expert_W):
    del router_W

    def body(x_ref, ridx_ref, ew_ref, out_ref, comm_ref, send_sems, recv_sems):
        my = lax.axis_index("i")
        left = (my - 1) % N_DEV
        right = (my + 1) % N_DEV

        barrier_sem = pltpu.get_barrier_semaphore()
        for nbr in (left, right):
            pl.semaphore_signal(
                barrier_sem, inc=1,
                device_id=(nbr,), device_id_type=pl.DeviceIdType.MESH,
            )
        pl.semaphore_wait(barrier_sem, 2)

        ridx = ridx_ref[:, :]
        eids = lax.broadcasted_iota(jnp.int32, (TOKENS, E_GLOBAL), 1)
        onehot = (ridx == eids).astype(jnp.float32)
        row = lax.broadcasted_iota(jnp.int32, (TOKENS, TOKENS), 0)
        col = lax.broadcasted_iota(jnp.int32, (TOKENS, TOKENS), 1)
        tril = (col < row).astype(jnp.float32)
        prior = jnp.dot(tril, onehot, preferred_element_type=jnp.float32)
        rank = jnp.sum(prior * onehot, axis=1, keepdims=True).astype(jnp.int32)

        col_s = lax.broadcasted_iota(jnp.int32, (TOKENS, CROWS), 1)
        col_r = col_s % SLOT

        def sel(o):
            col_e = o * E_LOCAL + col_s // SLOT
            return ((ridx == col_e) & (rank == col_r) & (col_r < CAP)).astype(
                jnp.float32
            )

        t_my = sel(my)
        cx = lax.dot_general(
            t_my, x_ref[:, :],
            dimension_numbers=(((0,), (0,)), ((), ())),
            preferred_element_type=jnp.float32,
        )
        for k in range(E_LOCAL):
            comm_ref[0, k * SLOT:(k + 1) * SLOT, :] = jnp.dot(
                cx[k * SLOT:(k + 1) * SLOT, :], ew_ref[k],
                preferred_element_type=jnp.float32,
            )

        out_ref[:, :] = jnp.dot(
            t_my, comm_ref[0, :, :], preferred_element_type=jnp.float32
        )

        for h in range(N_DEV - 1):
            rdma = pltpu.make_async_remote_copy(
                src_ref=comm_ref.at[h],
                dst_ref=comm_ref.at[h + 1],
                send_sem=send_sems.at[h],
                recv_sem=recv_sems.at[h],
                device_id=(right,),
                device_id_type=pl.DeviceIdType.MESH,
            )
            rdma.start()
            rdma.wait()
            o = (my - h - 1) % N_DEV
            out_ref[:, :] += jnp.dot(
                sel(o), comm_ref[h + 1, :, :],
                preferred_element_type=jnp.float32,
            )

    return pl.pallas_call(
        body,
        out_shape=jax.ShapeDtypeStruct((TOKENS, D_HIDDEN), jnp.float32),
        in_specs=[
            pl.BlockSpec(memory_space=pltpu.VMEM),
            pl.BlockSpec(memory_space=pltpu.VMEM),
            pl.BlockSpec(memory_space=pltpu.VMEM),
        ],
        out_specs=pl.BlockSpec(memory_space=pltpu.VMEM),
        scratch_shapes=[
            pltpu.VMEM((N_DEV, CROWS, D_HIDDEN), jnp.float32),
            pltpu.SemaphoreType.DMA((N_DEV - 1,)),
            pltpu.SemaphoreType.DMA((N_DEV - 1,)),
        ],
        compiler_params=pltpu.CompilerParams(collective_id=0),
    )(x, route_idx, expert_W)


# device time: 33631 ns/iter; 5.4593x vs baseline; 1.4202x over previous
import jax
import jax.numpy as jnp
from jax import lax
from jax.experimental import pallas as pl
from jax.experimental.pallas import tpu as pltpu

N_DEV = 8
TOKENS = 1024
D_MODEL = 256
D_HIDDEN = 512
E_GLOBAL = 32
E_LOCAL = E_GLOBAL // N_DEV
CAP = 25
SLOT = 32
CROWS = E_LOCAL * SLOT


def kernel(x, router_W, route_idx, expert_W):
    del router_W

    def body(x_ref, ridx_ref, ew_ref, out_ref, comm_ref, send_sems, recv_sems):
        my = lax.axis_index("i")
        left = (my - 1) % N_DEV
        right = (my + 1) % N_DEV

        barrier_sem = pltpu.get_barrier_semaphore()
        for nbr in (left, right):
            pl.semaphore_signal(
                barrier_sem, inc=1,
                device_id=(nbr,), device_id_type=pl.DeviceIdType.MESH,
            )
        pl.semaphore_wait(barrier_sem, 2)

        ridx = ridx_ref[:, :]
        eids = lax.broadcasted_iota(jnp.int32, (TOKENS, E_GLOBAL), 1)
        onehot = (ridx == eids).astype(jnp.float32)
        row = lax.broadcasted_iota(jnp.int32, (TOKENS, TOKENS), 0)
        col = lax.broadcasted_iota(jnp.int32, (TOKENS, TOKENS), 1)
        tril = (col < row).astype(jnp.float32)
        prior = jnp.dot(tril, onehot, preferred_element_type=jnp.float32)
        rank = jnp.sum(prior * onehot, axis=1, keepdims=True).astype(jnp.int32)

        col_s = lax.broadcasted_iota(jnp.int32, (TOKENS, CROWS), 1)
        col_r = col_s % SLOT

        def sel(o):
            col_e = o * E_LOCAL + col_s // SLOT
            return ((ridx == col_e) & (rank == col_r) & (col_r < CAP)).astype(
                jnp.float32
            )

        t_my = sel(my)
        cx = lax.dot_general(
            t_my, x_ref[:, :],
            dimension_numbers=(((0,), (0,)), ((), ())),
            preferred_element_type=jnp.float32,
        )
        for k in range(E_LOCAL):
            comm_ref[0, k * SLOT:(k + 1) * SLOT, :] = jnp.dot(
                cx[k * SLOT:(k + 1) * SLOT, :], ew_ref[k],
                preferred_element_type=jnp.float32,
            ).astype(comm_ref.dtype)

        def scatter_add(o, slot):
            out_ref[:, :] += jnp.dot(
                sel(o), comm_ref[slot, :, :].astype(jnp.float32),
                preferred_element_type=jnp.float32,
            )

        def hop(h):
            rdma = pltpu.make_async_remote_copy(
                src_ref=comm_ref.at[h],
                dst_ref=comm_ref.at[h + 1],
                send_sem=send_sems.at[h],
                recv_sem=recv_sems.at[h],
                device_id=(right,),
                device_id_type=pl.DeviceIdType.MESH,
            )
            rdma.start()
            return rdma

        prev = hop(0)
        out_ref[:, :] = jnp.dot(
            t_my, comm_ref[0, :, :].astype(jnp.float32),
            preferred_element_type=jnp.float32,
        )
        for h in range(1, N_DEV - 1):
            prev.wait_recv()
            cur = hop(h)
            prev.wait_send()
            scatter_add((my - h) % N_DEV, h)
            prev = cur
        prev.wait_recv()
        prev.wait_send()
        scatter_add((my + 1) % N_DEV, N_DEV - 1)

    return pl.pallas_call(
        body,
        out_shape=jax.ShapeDtypeStruct((TOKENS, D_HIDDEN), jnp.float32),
        in_specs=[
            pl.BlockSpec(memory_space=pltpu.VMEM),
            pl.BlockSpec(memory_space=pltpu.VMEM),
            pl.BlockSpec(memory_space=pltpu.VMEM),
        ],
        out_specs=pl.BlockSpec(memory_space=pltpu.VMEM),
        scratch_shapes=[
            pltpu.VMEM((N_DEV, CROWS, D_HIDDEN), jnp.bfloat16),
            pltpu.SemaphoreType.DMA((N_DEV - 1,)),
            pltpu.SemaphoreType.DMA((N_DEV - 1,)),
        ],
        compiler_params=pltpu.CompilerParams(collective_id=0),
    )(x, route_idx, expert_W)


# device time: 17310 ns/iter; 10.6067x vs baseline; 1.9429x over previous
import jax
import jax.numpy as jnp
from jax import lax
from jax.experimental import pallas as pl
from jax.experimental.pallas import tpu as pltpu

N_DEV = 8
TOKENS = 1024
D_MODEL = 256
D_HIDDEN = 512
E_GLOBAL = 32
E_LOCAL = E_GLOBAL // N_DEV
CAP = 25
SLOT = 32
CROWS = E_LOCAL * SLOT
PROWS = 104


def kernel(x, router_W, route_idx, expert_W):
    del router_W

    def body(x_ref, ridx_ref, ew_ref, out_ref,
             myblock, comm_ref, send_sems, recv_sems, copy_sem):
        my = lax.axis_index("i")

        barrier_sem = pltpu.get_barrier_semaphore()
        for j in range(1, N_DEV):
            pl.semaphore_signal(
                barrier_sem, inc=1,
                device_id=((my + j) % N_DEV,),
                device_id_type=pl.DeviceIdType.MESH,
            )

        BLK = 256
        NB = TOKENS // BLK
        ridx = ridx_ref[:, :]
        eids = lax.broadcasted_iota(jnp.int32, (TOKENS, E_GLOBAL), 1)
        is_e = ridx == eids
        onehot = is_e.astype(jnp.bfloat16)
        r_b = lax.broadcasted_iota(jnp.int32, (BLK, BLK), 0)
        c_b = lax.broadcasted_iota(jnp.int32, (BLK, BLK), 1)
        tril_b = (c_b < r_b).astype(jnp.bfloat16)
        intra = []
        offs = [jnp.zeros((1, E_GLOBAL), jnp.float32)]
        for b in range(NB):
            ob = onehot[b * BLK:(b + 1) * BLK, :]
            intra.append(
                jnp.dot(tril_b, ob, preferred_element_type=jnp.float32)
                + offs[-1]
            )
            offs.append(
                offs[-1]
                + jnp.sum(ob.astype(jnp.float32), axis=0, keepdims=True)
            )
        prior = jnp.concatenate(intra, axis=0)
        rank = jnp.sum(
            jnp.where(is_e, prior, 0.0), axis=1, keepdims=True
        ).astype(jnp.int32)

        ones8 = jnp.ones((8, 1), jnp.float32)
        ridx_t = lax.dot_general(
            ones8, ridx.astype(jnp.float32),
            dimension_numbers=(((1,), (1,)), ((), ())),
            preferred_element_type=jnp.float32,
        )[0:1, :].astype(jnp.int32)
        rank_t = lax.dot_general(
            ones8, rank.astype(jnp.float32),
            dimension_numbers=(((1,), (1,)), ((), ())),
            preferred_element_type=jnp.float32,
        )[0:1, :].astype(jnp.int32)

        prow = lax.broadcasted_iota(jnp.int32, (PROWS, TOKENS), 0)
        packed_si = (
            (ridx_t == my * E_LOCAL + prow // CAP)
            & (rank_t == prow % CAP)
            & (prow < E_LOCAL * CAP)
        ).astype(jnp.bfloat16)

        cx = jnp.dot(
            packed_si, x_ref[:, :].astype(jnp.bfloat16),
            preferred_element_type=jnp.float32,
        )
        crow = lax.broadcasted_iota(jnp.int32, (PROWS, E_LOCAL * D_MODEL), 0)
        ccol = lax.broadcasted_iota(jnp.int32, (PROWS, E_LOCAL * D_MODEL), 1)
        cx_bd = jnp.where(
            ccol // D_MODEL == crow // CAP,
            jnp.concatenate([cx] * E_LOCAL, axis=1),
            0.0,
        ).astype(jnp.bfloat16)
        w_stack = ew_ref[:, :, :].reshape(
            E_LOCAL * D_MODEL, D_HIDDEN
        ).astype(jnp.bfloat16)
        myblock[:, :] = jnp.dot(
            cx_bd, w_stack, preferred_element_type=jnp.float32
        ).astype(myblock.dtype)

        local_copy = pltpu.make_async_copy(myblock, comm_ref.at[my], copy_sem)
        local_copy.start()

        pl.semaphore_wait(barrier_sem, N_DEV - 1)

        sends = []
        for j in range(1, N_DEV):
            rdma = pltpu.make_async_remote_copy(
                src_ref=myblock,
                dst_ref=comm_ref.at[my],
                send_sem=send_sems.at[j - 1],
                recv_sem=recv_sems.at[my],
                device_id=((my + j) % N_DEV,),
                device_id_type=pl.DeviceIdType.MESH,
            )
            rdma.start()
            sends.append(rdma)

        col_q = lax.broadcasted_iota(jnp.int32, (TOKENS, N_DEV * PROWS), 1)
        q_p = col_q % PROWS
        sel_all = (
            (ridx == (col_q // PROWS) * E_LOCAL + q_p // CAP)
            & (rank == q_p % CAP)
            & (q_p < E_LOCAL * CAP)
        ).astype(jnp.bfloat16)

        for j in range(1, N_DEV):
            o = (my - j) % N_DEV
            recv = pltpu.make_async_remote_copy(
                src_ref=myblock,
                dst_ref=comm_ref.at[o],
                send_sem=send_sems.at[j - 1],
                recv_sem=recv_sems.at[o],
                device_id=(o,),
                device_id_type=pl.DeviceIdType.MESH,
            )
            recv.wait_recv()
        local_copy.wait()

        out_ref[:, :] = jnp.dot(
            sel_all,
            comm_ref[:, :, :].reshape(N_DEV * PROWS, D_HIDDEN),
            preferred_element_type=jnp.float32,
        )

        for rdma in sends:
            rdma.wait_send()

    return pl.pallas_call(
        body,
        out_shape=jax.ShapeDtypeStruct((TOKENS, D_HIDDEN), jnp.float32),
        in_specs=[
            pl.BlockSpec(memory_space=pltpu.VMEM),
            pl.BlockSpec(memory_space=pltpu.VMEM),
            pl.BlockSpec(memory_space=pltpu.VMEM),
        ],
        out_specs=pl.BlockSpec(memory_space=pltpu.VMEM),
        scratch_shapes=[
            pltpu.VMEM((PROWS, D_HIDDEN), jnp.bfloat16),
            pltpu.VMEM((N_DEV, PROWS, D_HIDDEN), jnp.bfloat16),
            pltpu.SemaphoreType.DMA((N_DEV - 1,)),
            pltpu.SemaphoreType.DMA((N_DEV,)),
            pltpu.SemaphoreType.DMA,
        ],
        compiler_params=pltpu.CompilerParams(collective_id=0),
    )(x, route_idx, expert_W)
